# Initial kernel scaffold; baseline (speedup 1.0000x reference)
import jax
import jax.numpy as jnp
from jax.experimental import pallas as pl


def kernel(user_id, item_id, title_token, desc_token, W_user, W_item, W_title, W_desc, B_user, B_item, B_title, B_desc):
    raise NotImplementedError("write your pallas kernel here")


# speedup vs baseline: 46.2366x; 46.2366x over previous
import functools

import jax
import jax.numpy as jnp
from jax import lax
from jax.experimental import pallas as pl
from jax.experimental.pallas import tpu as pltpu, tpu_sc as plsc

NC = 2
NS = 16
LANES = 16


def _full(v):
  return jnp.full((LANES,), v, jnp.int32)


def _build_sc_kernel(B, L, F, T):
  NW = NC * NS
  assert B % NW == 0
  bw = B // NW
  assert bw % 8 == 0 and F % LANES == 0
  mesh = plsc.VectorSubcoreMesh(
      core_axis_name="c", subcore_axis_name="s",
      num_cores=NC, num_subcores=NS)

  @functools.partial(
      pl.kernel,
      out_type=jax.ShapeDtypeStruct((B,), jnp.float32),
      mesh=mesh,
      compiler_params=pltpu.CompilerParams(
          needs_layout_passes=False, use_tc_tiling_on_sc=False),
      scratch_types=[
          pltpu.VMEM((bw,), jnp.int32),
          pltpu.VMEM((bw,), jnp.int32),
          pltpu.VMEM((bw, L), jnp.int32),
          pltpu.VMEM((bw, L), jnp.int32),
          pltpu.VMEM((bw, F), jnp.float32),
          pltpu.VMEM((bw, F), jnp.float32),
          pltpu.VMEM((bw,), jnp.float32),
          pltpu.VMEM((bw,), jnp.float32),
          pltpu.VMEM((2 * F,), jnp.float32),
          pltpu.VMEM((2 * F,), jnp.float32),
          pltpu.VMEM((24,), jnp.float32),
          pltpu.VMEM((24,), jnp.float32),
          pltpu.VMEM((16 + 3 * F,), jnp.float32),
          pltpu.VMEM((bw,), jnp.float32),
          pltpu.SemaphoreType.DMA,
          pltpu.SemaphoreType.DMA,
          pltpu.SemaphoreType.DMA,
          pltpu.SemaphoreType.DMA,
      ],
  )
  def sc_kernel(uid_h, iid_h, ttok_h, dtok_h, wu_h, wi_h, wt_h, wd_h,
                bu_h, bi_h, bt_h, bd_h, out_h,
                uid_v, iid_v, tt_v, td_v, u_v, v_v, bu_v, bi_v,
                wt_v, wd_v, bt_v, bd_v, cst_v, out_v,
                sem_u, sem_i, sem_bu, sem_bi):
    wid = lax.axis_index("s") * NC + lax.axis_index("c")
    base = wid * bw

    pltpu.sync_copy(uid_h.at[pl.ds(base, bw)], uid_v)
    pltpu.sync_copy(iid_h.at[pl.ds(base, bw)], iid_v)
    cu = pltpu.async_copy(wu_h.at[uid_v], u_v, sem_u)
    ci = pltpu.async_copy(wi_h.at[iid_v], v_v, sem_i)
    cbu = pltpu.async_copy(bu_h.at[uid_v], bu_v, sem_bu)
    cbi = pltpu.async_copy(bi_h.at[iid_v], bi_v, sem_bi)

    pltpu.sync_copy(ttok_h.at[pl.ds(base, bw)], tt_v)
    pltpu.sync_copy(dtok_h.at[pl.ds(base, bw)], td_v)
    pltpu.sync_copy(wt_h.at[pl.ds(0, 2 * F)], wt_v)
    pltpu.sync_copy(wd_h.at[pl.ds(0, 2 * F)], wd_v)
    pltpu.sync_copy(bt_h.at[pl.ds(0, 8)], bt_v.at[pl.ds(16, 8)])
    pltpu.sync_copy(bd_h.at[pl.ds(0, 8)], bd_v.at[pl.ds(16, 8)])

    for h in range(F // LANES):
      wt0 = wt_v[pl.ds(h * LANES, LANES)]
      wt1 = wt_v[pl.ds(F + h * LANES, LANES)]
      wd0 = wd_v[pl.ds(h * LANES, LANES)]
      wd1 = wd_v[pl.ds(F + h * LANES, LANES)]
      cst_v[pl.ds(16 + h * LANES, LANES)] = wt1 - wt0
      cst_v[pl.ds(16 + F + h * LANES, LANES)] = wd1 - wd0
      cst_v[pl.ds(16 + 2 * F + h * LANES, LANES)] = float(T) * (wt0 + wd0)

    bt0 = plsc.load_gather(bt_v, [_full(16)])
    bt1 = plsc.load_gather(bt_v, [_full(17)])
    bd0 = plsc.load_gather(bd_v, [_full(16)])
    bd1 = plsc.load_gather(bd_v, [_full(17)])
    dbt = bt1 - bt0
    dbd = bd1 - bd0
    bconst = float(T) * (bt0 + bd0)

    lane = lax.iota(jnp.int32, LANES)

    def distinct(tok_ref, g):
      row = _full(g * LANES) + lane
      tv = [plsc.load_gather(tok_ref, [row, _full(i)]) for i in range(L)]
      cnt = _full(0)
      for j in range(1, L):
        m = tv[0] == tv[j]
        for i in range(1, j):
          m = jnp.logical_or(m, tv[i] == tv[j])
        cnt = cnt + m.astype(jnp.int32)
      return (float(L) - cnt.astype(jnp.float32))

    nts = [distinct(tt_v, g) for g in range(bw // LANES)]
    nds = [distinct(td_v, g) for g in range(bw // LANES)]

    cu.wait()
    ci.wait()
    cbu.wait()
    cbi.wait()

    for g in range(bw // LANES):
      nt, nd = nts[g], nds[g]
      row = _full(g * LANES) + lane
      acc = jnp.zeros((LANES,), jnp.float32)
      for f in range(F):
        uf = plsc.load_gather(u_v, [row, _full(f)])
        vf = plsc.load_gather(v_v, [row, _full(f)])
        dtf = plsc.load_gather(cst_v, [_full(16 + f)])
        ddf = plsc.load_gather(cst_v, [_full(16 + F + f)])
        cf = plsc.load_gather(cst_v, [_full(16 + 2 * F + f)])
        acc = acc + uf * (vf + nt * dtf + nd * ddf + cf)
      res = (acc + bu_v[pl.ds(g * LANES, LANES)] + bi_v[pl.ds(g * LANES, LANES)]
             + nt * dbt + nd * dbd + bconst)
      out_v[pl.ds(g * LANES, LANES)] = res

    pltpu.sync_copy(out_v, out_h.at[pl.ds(base, bw)])

  return sc_kernel


def kernel(user_id, item_id, title_token, desc_token,
           W_user, W_item, W_title, W_desc,
           B_user, B_item, B_title, B_desc):
  B, L = title_token.shape
  F = W_user.shape[1]
  T = W_title.shape[0]
  sc = _build_sc_kernel(B, L, F, T)
  out = sc(user_id.reshape(B), item_id.reshape(B),
           title_token, desc_token,
           W_user, W_item,
           W_title.reshape(-1), W_desc.reshape(-1),
           B_user.reshape(-1), B_item.reshape(-1),
           B_title.reshape(-1), B_desc.reshape(-1))
  return out.reshape(B, 1)
